# Initial kernel scaffold; baseline (speedup 1.0000x reference)
import jax
import jax.numpy as jnp
from jax.experimental import pallas as pl


def kernel(x, y):
    raise NotImplementedError("write your pallas kernel here")


# speedup vs baseline: 2.2949x; 2.2949x over previous
import functools

import jax
import jax.numpy as jnp
from jax.experimental import pallas as pl

_LANES = 128


def _cmpx_rows(v, j, k, rows):
    t = 1 << (j - 7)
    g = rows // (2 * t)
    v5 = v.reshape(2, g, 2, t, _LANES)
    part = jnp.concatenate([v5[:, :, 1:2], v5[:, :, 0:1]], axis=2)
    part = part.reshape(2, rows, _LANES)
    mn = jnp.minimum(v, part)
    mx = jnp.maximum(v, part)
    row = jax.lax.broadcasted_iota(jnp.int32, (rows, 1), 0)
    bj = (row >> (j - 7)) & 1
    bk = (row >> (k - 7)) & 1
    keep_min = (bj == bk)[None]
    return jnp.where(keep_min, mn, mx)


def _cmpx_lanes(v, j, k, rows):
    s = 1 << j
    lane = jax.lax.broadcasted_iota(jnp.int32, (1, _LANES), 1)
    bj = (lane >> j) & 1
    down = jnp.roll(v, -s, axis=2)
    up = jnp.roll(v, s, axis=2)
    part = jnp.where((bj == 0)[None], down, up)
    mn = jnp.minimum(v, part)
    mx = jnp.maximum(v, part)
    if k >= 7:
        row = jax.lax.broadcasted_iota(jnp.int32, (rows, 1), 0)
        bk = (row >> (k - 7)) & 1
        keep_min = (bj == bk)[None]
    else:
        bk = (lane >> k) & 1
        keep_min = (bj == bk)[None]
    return jnp.where(keep_min, mn, mx)


def _emd_body(x_ref, y_ref, o_ref, *, rows, nreal, log2n):
    v = jnp.concatenate([x_ref[...], y_ref[...]], axis=0)
    for k in range(1, log2n + 1):
        for j in range(k - 1, -1, -1):
            if j >= 7:
                v = _cmpx_rows(v, j, k, rows)
            else:
                v = _cmpx_lanes(v, j, k, rows)
    row = jax.lax.broadcasted_iota(jnp.int32, (rows, _LANES), 0)
    lane = jax.lax.broadcasted_iota(jnp.int32, (rows, _LANES), 1)
    idx = row * _LANES + lane
    d = v[0] - v[1]
    d = jnp.where(idx < nreal, d, 0.0)
    o_ref[0] = jnp.full((8, _LANES), jnp.sum(d * d), jnp.float32)


def _emd_call(xp, yp, rows, nreal, log2n):
    b = xp.shape[0]
    body = functools.partial(_emd_body, rows=rows, nreal=nreal, log2n=log2n)
    return pl.pallas_call(
        body,
        grid=(b,),
        in_specs=[
            pl.BlockSpec((1, rows, _LANES), lambda i: (i, 0, 0)),
            pl.BlockSpec((1, rows, _LANES), lambda i: (i, 0, 0)),
        ],
        out_specs=pl.BlockSpec((1, 8, _LANES), lambda i: (i, 0, 0)),
        out_shape=jax.ShapeDtypeStruct((b, 8, _LANES), jnp.float32),
    )(xp, yp)


def kernel(x, y):
    b = x.shape[0]
    n = x.shape[1] * x.shape[2]
    log2n = max(8, (n - 1).bit_length())
    npad = 1 << log2n
    rows = npad // _LANES
    xf = x.reshape(b, n)
    yf = y.reshape(b, n)
    pad = npad - n
    xp = jnp.pad(xf, ((0, 0), (0, pad)), constant_values=jnp.inf)
    yp = jnp.pad(yf, ((0, 0), (0, pad)), constant_values=jnp.inf)
    xp = xp.reshape(b, rows, _LANES)
    yp = yp.reshape(b, rows, _LANES)
    out = _emd_call(xp, yp, rows, n, log2n)
    return jnp.sum(out[:, 0, 0]) / (b * n)


# speedup vs baseline: 3.9909x; 1.7390x over previous
import functools

import jax
import jax.numpy as jnp
from jax.experimental import pallas as pl

_LANES = 128


def _cmpx_row(v, t, keep_min):
    l, rows, _ = v.shape
    v5 = v.reshape(l, rows // (2 * t), 2, t, _LANES)
    part = jnp.concatenate([v5[:, :, 1:2], v5[:, :, 0:1]], axis=2)
    part = part.reshape(l, rows, _LANES)
    mn = jnp.minimum(v, part)
    mx = jnp.maximum(v, part)
    return jnp.where(keep_min, mn, mx)


def _row_iota(rows):
    return jax.lax.broadcasted_iota(jnp.int32, (rows, 1), 0)


def _lane_iota():
    return jax.lax.broadcasted_iota(jnp.int32, (1, _LANES), 1)


def _sort_chunks(v):
    desc = (jax.lax.broadcasted_iota(jnp.int32, (6, 1, 1), 0) % 3) != 0
    rowi = _row_iota(128)
    lanei = _lane_iota()
    vt = jnp.swapaxes(v, 1, 2)
    for k in range(1, 15):
        if k >= 8:
            v = jnp.swapaxes(vt, 1, 2)
            for j in range(k - 1, 6, -1):
                bj = (rowi >> (j - 7)) & 1
                bk = (rowi >> (k - 7)) & 1
                keep = ((bj == bk)[None] != desc)
                v = _cmpx_row(v, 1 << (j - 7), keep)
            vt = jnp.swapaxes(v, 1, 2)
        for j in range(min(k - 1, 6), -1, -1):
            bj = (rowi >> j) & 1
            if k < 7:
                bk = (rowi >> k) & 1
            else:
                bk = (lanei >> (k - 7)) & 1
            keep = ((bj == bk)[None] != desc)
            vt = _cmpx_row(vt, 1 << j, keep)
    return vt


def _merge_asc(vt, log2n):
    l, rt, _ = vt.shape
    rows = rt
    rowi_t = _row_iota(rt)
    v = _block_swap(vt)
    rowi = _row_iota(rows)
    for j in range(log2n - 1, 6, -1):
        keep = (((rowi >> (j - 7)) & 1) == 0)[None]
        v = _cmpx_row(v, 1 << (j - 7), keep)
    vt = _block_swap(v)
    for j in range(6, -1, -1):
        keep = (((rowi_t >> j) & 1) == 0)[None]
        vt = _cmpx_row(vt, 1 << j, keep)
    return vt


def _block_swap(v):
    l, rows, _ = v.shape
    nblk = rows // 128
    v4 = v.reshape(l, nblk, 128, _LANES)
    v4 = jnp.swapaxes(v4, 2, 3)
    return v4.reshape(l, rows, _LANES)


def _emd_body(x_ref, y_ref, o_ref, *, nreal):
    v = jnp.concatenate([x_ref[...], y_ref[...]], axis=0)
    v = v.reshape(2, 3, 128, _LANES).reshape(6, 128, _LANES)
    vt = _sort_chunks(v)
    vt = vt.reshape(2, 3, 128, _LANES)
    mt = vt[:, 0:2].reshape(2, 256, _LANES)
    mt = _merge_asc(mt, 15)
    infs = jnp.full((2, 1, 128, _LANES), jnp.inf, jnp.float32)
    wt = jnp.concatenate(
        [mt.reshape(2, 2, 128, _LANES), infs, vt[:, 2:3]], axis=1
    ).reshape(2, 512, _LANES)
    wt = _merge_asc(wt, 16)
    d = wt[0, :384] - wt[1, :384]
    o_ref[0] = jnp.full((8, _LANES), jnp.sum(d * d), jnp.float32)


def _emd_call(xp, yp, nreal):
    b = xp.shape[0]
    body = functools.partial(_emd_body, nreal=nreal)
    return pl.pallas_call(
        body,
        grid=(b,),
        in_specs=[
            pl.BlockSpec((1, 384, _LANES), lambda i: (i, 0, 0)),
            pl.BlockSpec((1, 384, _LANES), lambda i: (i, 0, 0)),
        ],
        out_specs=pl.BlockSpec((1, 8, _LANES), lambda i: (i, 0, 0)),
        out_shape=jax.ShapeDtypeStruct((b, 8, _LANES), jnp.float32),
    )(xp, yp)


def kernel(x, y):
    b = x.shape[0]
    n = x.shape[1] * x.shape[2]
    xp = x.reshape(b, 384, _LANES)
    yp = y.reshape(b, 384, _LANES)
    out = _emd_call(xp, yp, n)
    return jnp.sum(out[:, 0, 0]) / (b * n)
